# baseline (speedup 1.0000x reference)
import functools
import math

import jax
import jax.numpy as jnp
import numpy as np
from jax import lax
from jax.experimental import pallas as pl
from jax.experimental.pallas import tpu as pltpu
from jax.experimental.pallas import tpu_sc as plsc

_TWO_PI = 2.0 * math.pi
_INV_2PI = 1.0 / _TWO_PI
_TPI_HI = float(np.float32(_TWO_PI))
_TPI_LO = _TWO_PI - _TPI_HI
_COS_COEF = (9.99999211e-01, -4.99994213e-01, 4.16597778e-02,
             -1.38587899e-03, 2.42029414e-05, -2.19729638e-07)
_SIN_COEF = (9.99999600e-01, -1.66665526e-01, 8.33240299e-03,
             -1.98086333e-04, 2.69971464e-06, -2.03622449e-08)


def _sc_gather_body(comb_hbm, idx_hbm, out_hbm, idx_v, comb_v, sem, *,
                    per_w, ch, n_chunks, nc):
    wid = lax.axis_index("s") * nc + lax.axis_index("c")
    base = wid * per_w
    for j in range(n_chunks):
        off = base + j * ch
        pltpu.sync_copy(idx_hbm.at[pl.ds(off, ch)], idx_v)
        pltpu.async_copy(comb_hbm.at[idx_v], comb_v, sem).wait()
        pltpu.sync_copy(comb_v, out_hbm.at[pl.ds(off, ch)])


def _sc_gather(comb, idx_flat):
    n_total = idx_flat.shape[0]
    d2 = comb.shape[1]
    info = plsc.get_sparse_core_info()
    nc, ns = info.num_cores, info.num_subcores
    nw = nc * ns
    per_w = n_total // nw
    ch = 320
    n_chunks = per_w // ch
    assert per_w % ch == 0 and n_total % nw == 0

    mesh = plsc.VectorSubcoreMesh(core_axis_name="c", subcore_axis_name="s")
    kern = pl.kernel(
        functools.partial(_sc_gather_body, per_w=per_w, ch=ch,
                          n_chunks=n_chunks, nc=nc),
        mesh=mesh,
        out_type=jax.ShapeDtypeStruct((n_total, d2), jnp.float32),
        scratch_types=[
            pltpu.VMEM((ch,), jnp.int32),
            pltpu.VMEM((ch, d2), jnp.float32),
            pltpu.SemaphoreType.DMA,
        ],
    )
    return kern(comb, idx_flat)


def _sincos_pack(c, e, pos, d):
    freq = c[:, :d]
    bias = c[:, d:]
    phase = pos * freq + bias
    n = jnp.round(phase * _INV_2PI)
    r = (phase - n * _TPI_HI) - n * _TPI_LO
    r2 = r * r
    rr2 = jnp.concatenate([r2, r2], axis=1)
    left = lax.broadcasted_iota(jnp.int32, (1, 2 * d), 1) < d
    poly = jnp.where(left, _COS_COEF[5], _SIN_COEF[5]).astype(jnp.float32)
    for k in (4, 3, 2, 1, 0):
        ck = jnp.where(left, _COS_COEF[k], _SIN_COEF[k]).astype(jnp.float32)
        poly = poly * rr2 + ck
    m = jnp.concatenate([e, e * r], axis=1)
    return m * poly


def _tc_body_first(c_ref, e_ref, real_ref, imag_ref, *, blk, b_per_s, d,
                   row_off):
    i = pl.program_id(0)
    pos = ((lax.broadcasted_iota(jnp.int32, (blk, d), 0) + row_off + i * blk)
           // b_per_s + 1).astype(jnp.float32)
    out = _sincos_pack(c_ref[...], e_ref[...].reshape(blk, d), pos, d)
    real_ref[...] = out[:, :d]
    imag_ref[...] = out[:, d:]


def _tc_body_chain(rp_ref, ip_ref, c_ref, e_ref, real_ref, imag_ref, *, blk,
                   b_per_s, d, row_off):
    del rp_ref, ip_ref
    _tc_body_first(c_ref, e_ref, real_ref, imag_ref, blk=blk,
                   b_per_s=b_per_s, d=d, row_off=row_off)


def _tc_slice(comb_k, emb, k, n_slices, prev):
    s, b_per_s, d = emb.shape
    n = s * b_per_s
    blk = 2048
    s_blk = blk // b_per_s
    steps = n // blk // n_slices
    off = k * steps
    out_shape = [jax.ShapeDtypeStruct((n, d), jnp.float32)] * 2
    spec_c = pl.BlockSpec((blk, 2 * d), lambda i: (i, 0))
    spec_off = pl.BlockSpec((blk, d), lambda i: (i + off, 0))
    spec_e = pl.BlockSpec((s_blk, b_per_s, d), lambda i: (i + off, 0, 0))
    kwargs = dict(grid=(steps,), out_specs=[spec_off, spec_off],
                  out_shape=out_shape)
    body_kw = dict(blk=blk, b_per_s=b_per_s, d=d, row_off=off * blk)
    if prev is None:
        return pl.pallas_call(
            functools.partial(_tc_body_first, **body_kw),
            in_specs=[spec_c, spec_e], **kwargs,
        )(comb_k, emb)
    spec_any = pl.BlockSpec(memory_space=pltpu.MemorySpace.HBM)
    return pl.pallas_call(
        functools.partial(_tc_body_chain, **body_kw),
        in_specs=[spec_any, spec_any, spec_c, spec_e],
        input_output_aliases={0: 0, 1: 1},
        **kwargs,
    )(prev[0], prev[1], comb_k, emb)


def kernel(emb, x, freq_table, phase_table):
    s, b, d = emb.shape
    comb = jnp.concatenate([freq_table, phase_table], axis=1)
    n_slices = 5
    s_sl = s // n_slices
    prev = None
    for k in range(n_slices):
        idx_k = x[k * s_sl:(k + 1) * s_sl].reshape(-1)
        comb_k = _sc_gather(comb, idx_k)
        prev = _tc_slice(comb_k, emb, k, n_slices, prev)
    real, imag = prev
    return real.reshape(s, b, d), imag.reshape(s, b, d)


# speedup vs baseline: 1.1010x; 1.1010x over previous
import functools
import math

import jax
import jax.numpy as jnp
import numpy as np
from jax import lax
from jax.experimental import pallas as pl
from jax.experimental.pallas import tpu as pltpu
from jax.experimental.pallas import tpu_sc as plsc

_TWO_PI = 2.0 * math.pi
_INV_2PI = 1.0 / _TWO_PI
_TPI_HI = float(np.float32(_TWO_PI))
_TPI_LO = _TWO_PI - _TPI_HI
_COS_COEF = (9.99999211e-01, -4.99994213e-01, 4.16597778e-02,
             -1.38587899e-03, 2.42029414e-05, -2.19729638e-07)
_SIN_COEF = (9.99999600e-01, -1.66665526e-01, 8.33240299e-03,
             -1.98086333e-04, 2.69971464e-06, -2.03622449e-08)


def _sc_gather_body(comb_hbm, idx_hbm, out_hbm, idx_v, comb_v, sem, *,
                    per_w, ch, n_chunks, nc):
    wid = lax.axis_index("s") * nc + lax.axis_index("c")
    base = wid * per_w
    for j in range(n_chunks):
        off = base + j * ch
        pltpu.sync_copy(idx_hbm.at[pl.ds(off, ch)], idx_v)
        pltpu.async_copy(comb_hbm.at[idx_v], comb_v, sem).wait()
        pltpu.sync_copy(comb_v, out_hbm.at[pl.ds(off, ch)])


def _sc_gather(comb, idx_flat):
    n_total = idx_flat.shape[0]
    d2 = comb.shape[1]
    info = plsc.get_sparse_core_info()
    nc, ns = info.num_cores, info.num_subcores
    nw = nc * ns
    per_w = n_total // nw
    ch = 320
    n_chunks = per_w // ch
    assert per_w % ch == 0 and n_total % nw == 0

    mesh = plsc.VectorSubcoreMesh(core_axis_name="c", subcore_axis_name="s")
    kern = pl.kernel(
        functools.partial(_sc_gather_body, per_w=per_w, ch=ch,
                          n_chunks=n_chunks, nc=nc),
        mesh=mesh,
        out_type=jax.ShapeDtypeStruct((n_total, d2), jnp.float32),
        scratch_types=[
            pltpu.VMEM((ch,), jnp.int32),
            pltpu.VMEM((ch, d2), jnp.float32),
            pltpu.SemaphoreType.DMA,
        ],
    )
    return kern(comb, idx_flat)


def _sincos_pack(c, e, pos, d):
    freq = c[:, :d]
    bias = c[:, d:]
    phase = pos * freq + bias
    n = jnp.round(phase * _INV_2PI)
    r = (phase - n * _TPI_HI) - n * _TPI_LO
    r2 = r * r
    rr2 = jnp.concatenate([r2, r2], axis=1)
    left = lax.broadcasted_iota(jnp.int32, (1, 2 * d), 1) < d
    poly = jnp.where(left, _COS_COEF[5], _SIN_COEF[5]).astype(jnp.float32)
    for k in (4, 3, 2, 1, 0):
        ck = jnp.where(left, _COS_COEF[k], _SIN_COEF[k]).astype(jnp.float32)
        poly = poly * rr2 + ck
    m = jnp.concatenate([e, e * r], axis=1)
    return m * poly


def _tc_body_first(c_ref, e_ref, real_ref, imag_ref, *, blk, b_per_s, d,
                   row_off):
    i = pl.program_id(0)
    pos = ((lax.broadcasted_iota(jnp.int32, (blk, d), 0) + row_off + i * blk)
           // b_per_s + 1).astype(jnp.float32)
    out = _sincos_pack(c_ref[...], e_ref[...], pos, d)
    real_ref[...] = out[:, :d]
    imag_ref[...] = out[:, d:]


def _tc_body_chain(rp_ref, ip_ref, c_ref, e_ref, real_ref, imag_ref, *, blk,
                   b_per_s, d, row_off):
    del rp_ref, ip_ref
    _tc_body_first(c_ref, e_ref, real_ref, imag_ref, blk=blk,
                   b_per_s=b_per_s, d=d, row_off=row_off)


def _tc_slice(comb_k, e_flat, k, n_slices, prev):
    n, d = e_flat.shape
    blk = 4096
    steps = n // blk // n_slices
    off = k * steps
    b_per_s = 1024
    out_shape = [jax.ShapeDtypeStruct((n, d), jnp.float32)] * 2
    spec_c = pl.BlockSpec((blk, 2 * d), lambda i: (i, 0))
    spec_off = pl.BlockSpec((blk, d), lambda i: (i + off, 0))
    kwargs = dict(grid=(steps,), out_specs=[spec_off, spec_off],
                  out_shape=out_shape)
    body_kw = dict(blk=blk, b_per_s=b_per_s, d=d, row_off=off * blk)
    if prev is None:
        return pl.pallas_call(
            functools.partial(_tc_body_first, **body_kw),
            in_specs=[spec_c, spec_off], **kwargs,
        )(comb_k, e_flat)
    spec_any = pl.BlockSpec(memory_space=pltpu.MemorySpace.HBM)
    return pl.pallas_call(
        functools.partial(_tc_body_chain, **body_kw),
        in_specs=[spec_any, spec_any, spec_c, spec_off],
        input_output_aliases={0: 0, 1: 1},
        **kwargs,
    )(prev[0], prev[1], comb_k, e_flat)


def kernel(emb, x, freq_table, phase_table):
    s, b, d = emb.shape
    comb = jnp.concatenate([freq_table, phase_table], axis=1)
    n_slices = 5
    s_sl = s // n_slices
    e_flat = emb.reshape(s * b, d)
    prev = None
    for k in range(n_slices):
        idx_k = x[k * s_sl:(k + 1) * s_sl].reshape(-1)
        comb_k = _sc_gather(comb, idx_k)
        prev = _tc_slice(comb_k, e_flat, k, n_slices, prev)
    real, imag = prev
    return real.reshape(s, b, d), imag.reshape(s, b, d)
